# baseline (speedup 1.0000x reference)
import functools

import jax
import jax.numpy as jnp
from jax import lax
from jax.experimental import pallas as pl
from jax.experimental.pallas import tpu as pltpu
from jax.experimental.pallas import tpu_sc as plsc

NC = 2
NS = 16
NW = NC * NS

NBUF = 4
TILE = 128
SLOTS = 4


def _gather_kernel(n_tiles, d, idx_hbm, table_hbm, out_hbm, table_v, idx_v,
                   idx_t_v, rows_v, isem, gsem, ssem):
    sid = lax.axis_index("s")
    wid = sid * NC + lax.axis_index("c")
    row0 = wid * n_tiles * TILE
    n_groups = n_tiles // NBUF

    @pl.when(sid == 0)
    def _():
        pltpu.sync_copy(table_hbm, table_v)

    plsc.subcore_barrier()

    lanes = lax.iota(jnp.int32, 16)

    def start_idx(i, b):
        pltpu.async_copy(
            idx_hbm.at[pl.ds((wid * n_tiles + i) * SLOTS, SLOTS)],
            idx_v.at[b], isem.at[b])

    def wait_idx(b):
        pltpu.make_async_copy(
            idx_hbm.at[pl.ds(0, SLOTS)], idx_v.at[b], isem.at[b]).wait()

    def transpose_idx(b):
        for q in range(SLOTS):
            for k in range(TILE // 16):
                rows = jnp.full((16,), k // 2, jnp.int32)
                cols = SLOTS * lanes + (64 * (k % 2) + q)
                vals = plsc.load_gather(idx_v.at[b], [rows, cols])
                idx_t_v[b, q, pl.ds(16 * k, 16)] = vals

    def fire_gathers(b):
        for q in range(SLOTS):
            pltpu.async_copy(
                table_v.at[idx_t_v.at[b].at[q]],
                rows_v.at[b].at[q],
                gsem.at[b])

    def wait_gathers(b):
        pltpu.make_async_copy(
            out_hbm.at[pl.ds(0, SLOTS * TILE), pl.ds(0, d)], rows_v.at[b],
            gsem.at[b]).wait()

    def start_store(i, b):
        for q in range(SLOTS):
            pltpu.async_copy(
                rows_v.at[b].at[q],
                out_hbm.at[pl.ds(row0 + i * TILE, TILE), pl.ds(q * d, d)],
                ssem.at[b])

    def wait_store(b):
        pltpu.make_async_copy(
            out_hbm.at[pl.ds(0, SLOTS * TILE), pl.ds(0, d)], rows_v.at[b],
            ssem.at[b]).wait()

    for b in range(NBUF):
        start_idx(b, b)
    for b in range(NBUF):
        wait_idx(b)
        transpose_idx(b)
        fire_gathers(b)

    def group_body(g, carry):
        i0 = g * NBUF
        for b in range(NBUF):
            wait_gathers(b)
            start_store(i0 + b, b)
            start_idx(i0 + NBUF + b, b)
        for b in range(NBUF):
            wait_store(b)
            wait_idx(b)
            transpose_idx(b)
            fire_gathers(b)
        return carry

    lax.fori_loop(0, n_groups - 1, group_body, 0)

    i0 = (n_groups - 1) * NBUF
    for b in range(NBUF):
        wait_gathers(b)
        start_store(i0 + b, b)
    for b in range(NBUF):
        wait_store(b)


def kernel(data, embed_table):
    b, l, s = data.shape
    n_rows = b * l
    d = embed_table.shape[1]
    assert s == SLOTS and s * d == 128
    assert n_rows % (NW * TILE * NBUF) == 0
    n_tiles = n_rows // (NW * TILE)

    idx_t = data.reshape(n_rows * s // 128, 128)

    mesh = plsc.VectorSubcoreMesh(
        core_axis_name="c", subcore_axis_name="s",
        num_cores=NC, num_subcores=NS,
    )
    run = pl.kernel(
        functools.partial(_gather_kernel, n_tiles, d),
        out_type=jax.ShapeDtypeStruct((n_rows, s * d), jnp.float32),
        mesh=mesh,
        scratch_types=[
            pltpu.VMEM_SHARED((133, d), jnp.float32),
            pltpu.VMEM((NBUF, SLOTS, TILE), jnp.int32),
            pltpu.VMEM((NBUF, SLOTS, TILE), jnp.int32),
            pltpu.VMEM((NBUF, SLOTS, TILE, d), jnp.float32),
            pltpu.SemaphoreType.DMA((NBUF,)),
            pltpu.SemaphoreType.DMA((NBUF,)),
            pltpu.SemaphoreType.DMA((NBUF,)),
        ],
        compiler_params=pltpu.CompilerParams(use_tc_tiling_on_sc=False, needs_layout_passes=False),
    )
    out = run(idx_t, embed_table)
    return out.reshape(b, l, s * d)


# speedup vs baseline: 3.7685x; 3.7685x over previous
import functools

import jax
import jax.numpy as jnp
from jax import lax
from jax.experimental import pallas as pl
from jax.experimental.pallas import tpu as pltpu
from jax.experimental.pallas import tpu_sc as plsc

NC = 2
NS = 16
NW = NC * NS

NBUF = 2
TILE = 256
SLOTS = 4


def _gather_kernel(n_tiles, d, idx_hbm, table_hbm, out_hbm, table_v, idx_v,
                   rows_v, isem, gsem, ssem):
    sid = lax.axis_index("s")
    wid = sid * NC + lax.axis_index("c")
    row0 = wid * n_tiles * TILE
    n_groups = n_tiles // NBUF

    @pl.when(sid == 0)
    def _():
        pltpu.sync_copy(table_hbm, table_v)

    plsc.subcore_barrier()

    def start_idx(i, b):
        pltpu.async_copy(
            idx_hbm.at[:, pl.ds(row0 + i * TILE, TILE)],
            idx_v.at[b], isem.at[b])

    def wait_idx(b):
        pltpu.make_async_copy(
            idx_hbm.at[:, pl.ds(0, TILE)], idx_v.at[b], isem.at[b]).wait()

    def fire_gathers(b):
        for q in range(SLOTS):
            for k in range(TILE // 128):
                pltpu.async_copy(
                    table_v.at[idx_v.at[b].at[q].at[pl.ds(128 * k, 128)]],
                    rows_v.at[b].at[q].at[pl.ds(128 * k, 128)],
                    gsem.at[b])

    def wait_gathers(b):
        pltpu.make_async_copy(
            out_hbm.at[pl.ds(0, SLOTS * TILE), pl.ds(0, d)], rows_v.at[b],
            gsem.at[b]).wait()

    def start_store(i, b):
        for q in range(SLOTS):
            pltpu.async_copy(
                rows_v.at[b].at[q],
                out_hbm.at[pl.ds(row0 + i * TILE, TILE), pl.ds(q * d, d)],
                ssem.at[b])

    def wait_store(b):
        pltpu.make_async_copy(
            out_hbm.at[pl.ds(0, SLOTS * TILE), pl.ds(0, d)], rows_v.at[b],
            ssem.at[b]).wait()

    for b in range(NBUF):
        start_idx(b, b)
    for b in range(NBUF):
        wait_idx(b)
        fire_gathers(b)

    def group_body(g, carry):
        i0 = g * NBUF
        for b in range(NBUF):
            wait_gathers(b)
            start_store(i0 + b, b)
            start_idx(i0 + NBUF + b, b)
        for b in range(NBUF):
            wait_store(b)
            wait_idx(b)
            fire_gathers(b)
        return carry

    lax.fori_loop(0, n_groups - 1, group_body, 0)

    i0 = (n_groups - 1) * NBUF
    for b in range(NBUF):
        wait_gathers(b)
        start_store(i0 + b, b)
    for b in range(NBUF):
        wait_store(b)


def kernel(data, embed_table):
    b, l, s = data.shape
    n_rows = b * l
    d = embed_table.shape[1]
    assert s == SLOTS and s * d == 128
    assert n_rows % (NW * TILE * NBUF) == 0
    n_tiles = n_rows // (NW * TILE)

    idx_t = data.reshape(n_rows, s).T

    mesh = plsc.VectorSubcoreMesh(
        core_axis_name="c", subcore_axis_name="s",
        num_cores=NC, num_subcores=NS,
    )
    run = pl.kernel(
        functools.partial(_gather_kernel, n_tiles, d),
        out_type=jax.ShapeDtypeStruct((n_rows, s * d), jnp.float32),
        mesh=mesh,
        scratch_types=[
            pltpu.VMEM_SHARED((133, d), jnp.float32),
            pltpu.VMEM((NBUF, SLOTS, TILE), jnp.int32),
            pltpu.VMEM((NBUF, SLOTS, TILE, d), jnp.float32),
            pltpu.SemaphoreType.DMA((NBUF,)),
            pltpu.SemaphoreType.DMA((NBUF,)),
            pltpu.SemaphoreType.DMA((NBUF,)),
        ],
        compiler_params=pltpu.CompilerParams(use_tc_tiling_on_sc=False),
    )
    out = run(idx_t, embed_table)
    return out.reshape(b, l, s * d)


# speedup vs baseline: 4.9436x; 1.3118x over previous
import functools

import jax
import jax.numpy as jnp
from jax import lax
from jax.experimental import pallas as pl
from jax.experimental.pallas import tpu as pltpu
from jax.experimental.pallas import tpu_sc as plsc

NC = 2
NS = 16
NW = NC * NS

NBUF = 5
TILE = 128
SLOTS = 4


def _gather_kernel(n_tiles, d, idx_hbm, table_hbm, out_hbm, table_v, idx_v,
                   rows_v, isem, gsem, ssem):
    sid = lax.axis_index("s")
    wid = sid * NC + lax.axis_index("c")
    row0 = wid * n_tiles * TILE
    n_groups = n_tiles // NBUF

    @pl.when(sid == 0)
    def _():
        pltpu.sync_copy(table_hbm, table_v)

    plsc.subcore_barrier()

    def start_idx(i, b):
        pltpu.async_copy(
            idx_hbm.at[:, pl.ds(row0 + i * TILE, TILE)],
            idx_v.at[b], isem.at[b])

    def wait_idx(b):
        pltpu.make_async_copy(
            idx_hbm.at[:, pl.ds(0, TILE)], idx_v.at[b], isem.at[b]).wait()

    def fire_gathers(b):
        for q in range(SLOTS):
            pltpu.async_copy(
                table_v.at[idx_v.at[b].at[q]],
                rows_v.at[b].at[q],
                gsem.at[b])

    def wait_gathers(b):
        pltpu.make_async_copy(
            out_hbm.at[pl.ds(0, SLOTS * TILE), pl.ds(0, d)], rows_v.at[b],
            gsem.at[b]).wait()

    def start_store(i, b):
        for q in range(SLOTS):
            pltpu.async_copy(
                rows_v.at[b].at[q],
                out_hbm.at[pl.ds(row0 + i * TILE, TILE), pl.ds(q * d, d)],
                ssem.at[b])

    def wait_store(b):
        pltpu.make_async_copy(
            out_hbm.at[pl.ds(0, SLOTS * TILE), pl.ds(0, d)], rows_v.at[b],
            ssem.at[b]).wait()

    for b in range(NBUF):
        start_idx(b, b)
    for b in range(NBUF):
        wait_idx(b)
        fire_gathers(b)

    def group_body(g, carry):
        i0 = g * NBUF
        for b in range(NBUF):
            wait_gathers(b)
            start_store(i0 + b, b)
            start_idx(i0 + NBUF + b, b)
        for b in range(NBUF):
            wait_store(b)
            wait_idx(b)
            fire_gathers(b)
        return carry

    lax.fori_loop(0, n_groups - 1, group_body, 0)

    i0 = (n_groups - 1) * NBUF
    for b in range(NBUF):
        wait_gathers(b)
        start_store(i0 + b, b)
    for b in range(NBUF):
        wait_store(b)


def kernel(data, embed_table):
    b, l, s = data.shape
    n_rows = b * l
    d = embed_table.shape[1]
    assert s == SLOTS and s * d == 128
    assert n_rows % (NW * TILE * NBUF) == 0
    n_tiles = n_rows // (NW * TILE)

    idx_t = data.reshape(n_rows, s).T

    mesh = plsc.VectorSubcoreMesh(
        core_axis_name="c", subcore_axis_name="s",
        num_cores=NC, num_subcores=NS,
    )
    run = pl.kernel(
        functools.partial(_gather_kernel, n_tiles, d),
        out_type=jax.ShapeDtypeStruct((n_rows, s * d), jnp.float32),
        mesh=mesh,
        scratch_types=[
            pltpu.VMEM_SHARED((133, d), jnp.float32),
            pltpu.VMEM((NBUF, SLOTS, TILE), jnp.int32),
            pltpu.VMEM((NBUF, SLOTS, TILE, d), jnp.float32),
            pltpu.SemaphoreType.DMA((NBUF,)),
            pltpu.SemaphoreType.DMA((NBUF,)),
            pltpu.SemaphoreType.DMA((NBUF,)),
        ],
        compiler_params=pltpu.CompilerParams(use_tc_tiling_on_sc=False),
    )
    out = run(idx_t, embed_table)
    return out.reshape(b, l, s * d)


# speedup vs baseline: 4.9467x; 1.0006x over previous
import functools

import jax
import jax.numpy as jnp
from jax import lax
from jax.experimental import pallas as pl
from jax.experimental.pallas import tpu as pltpu
from jax.experimental.pallas import tpu_sc as plsc

NC = 2
NS = 16
NW = NC * NS

NBUF = 4
TILE = 128
SLOTS = 4


def _gather_kernel(n_tiles, d, idx_hbm, table_hbm, out_hbm, table_v, idx_v,
                   rows_v, isem, gsem, ssem):
    sid = lax.axis_index("s")
    wid = sid * NC + lax.axis_index("c")
    row0 = wid * n_tiles * TILE
    n_groups = n_tiles // NBUF

    @pl.when(sid == 0)
    def _():
        pltpu.sync_copy(table_hbm, table_v)

    plsc.subcore_barrier()

    def start_idx(i, b):
        pltpu.async_copy(
            idx_hbm.at[:, pl.ds(row0 + i * TILE, TILE)],
            idx_v.at[b], isem.at[b])

    def wait_idx(b):
        pltpu.make_async_copy(
            idx_hbm.at[:, pl.ds(0, TILE)], idx_v.at[b], isem.at[b]).wait()

    def fire_gathers(b):
        for q in range(SLOTS):
            pltpu.async_copy(
                table_v.at[idx_v.at[b].at[q]],
                rows_v.at[b].at[q],
                gsem.at[b])

    def wait_gathers(b):
        pltpu.make_async_copy(
            out_hbm.at[pl.ds(0, SLOTS * TILE), pl.ds(0, d)], rows_v.at[b],
            gsem.at[b]).wait()

    def start_store(i, b):
        for q in range(SLOTS):
            pltpu.async_copy(
                rows_v.at[b].at[q],
                out_hbm.at[pl.ds(row0 + i * TILE, TILE), pl.ds(q * d, d)],
                ssem.at[b])

    def wait_store(b):
        pltpu.make_async_copy(
            out_hbm.at[pl.ds(0, SLOTS * TILE), pl.ds(0, d)], rows_v.at[b],
            ssem.at[b]).wait()

    for b in range(NBUF):
        start_idx(b, b)
    for b in range(NBUF):
        wait_idx(b)
        fire_gathers(b)

    def group_body(g, carry):
        i0 = g * NBUF
        for b in range(NBUF):
            wait_gathers(b)
            start_store(i0 + b, b)
            start_idx(i0 + NBUF + b, b)
        for b in range(NBUF):
            wait_store(b)
            wait_idx(b)
            fire_gathers(b)
        return carry

    lax.fori_loop(0, n_groups - 1, group_body, 0)

    i0 = (n_groups - 1) * NBUF
    for b in range(NBUF):
        wait_gathers(b)
        start_store(i0 + b, b)
    for b in range(NBUF):
        wait_store(b)


def kernel(data, embed_table):
    b, l, s = data.shape
    n_rows = b * l
    d = embed_table.shape[1]
    assert s == SLOTS and s * d == 128
    assert n_rows % (NW * TILE * NBUF) == 0
    n_tiles = n_rows // (NW * TILE)

    idx_t = data.reshape(n_rows, s).T

    mesh = plsc.VectorSubcoreMesh(
        core_axis_name="c", subcore_axis_name="s",
        num_cores=NC, num_subcores=NS,
    )
    run = pl.kernel(
        functools.partial(_gather_kernel, n_tiles, d),
        out_type=jax.ShapeDtypeStruct((n_rows, s * d), jnp.float32),
        mesh=mesh,
        scratch_types=[
            pltpu.VMEM_SHARED((133, d), jnp.float32),
            pltpu.VMEM((NBUF, SLOTS, TILE), jnp.int32),
            pltpu.VMEM((NBUF, SLOTS, TILE, d), jnp.float32),
            pltpu.SemaphoreType.DMA((NBUF,)),
            pltpu.SemaphoreType.DMA((NBUF,)),
            pltpu.SemaphoreType.DMA((NBUF,)),
        ],
        compiler_params=pltpu.CompilerParams(use_tc_tiling_on_sc=False),
    )
    out = run(idx_t, embed_table)
    return out.reshape(b, l, s * d)
